# Initial kernel scaffold; baseline (speedup 1.0000x reference)
import jax
import jax.numpy as jnp
from jax.experimental import pallas as pl


def kernel(nodes, edges, receivers, senders, globals_, W, b):
    raise NotImplementedError("write your pallas kernel here")


# speedup vs baseline: 4.0976x; 4.0976x over previous
import functools

import jax
import jax.numpy as jnp
from jax import lax
from jax.experimental import pallas as pl
from jax.experimental.pallas import tpu as pltpu
from jax.experimental.pallas import tpu_sc as plsc

N_NODES = 10000
N_EDGES = 160000
D_FEAT = 256
D_EDGE = 16
D_GLOBAL = 128

NC = 2
NS = 16
NW = NC * NS
EDGES_PER_TILE = N_EDGES // NW
CHUNK = 125
CHUNKS_PER_TILE = EDGES_PER_TILE // CHUNK
N_PAD = 10240
ROWS_PER_TILE = N_PAD // NS


def _sc_segment_sum(edges, receivers2d, zeros):
    mesh = plsc.VectorSubcoreMesh(core_axis_name="c", subcore_axis_name="s")

    @functools.partial(
        pl.kernel,
        out_type=jax.ShapeDtypeStruct((NC, N_PAD, D_EDGE), jnp.float32),
        mesh=mesh,
        scratch_types=[
            pltpu.VMEM((EDGES_PER_TILE, D_EDGE), jnp.float32),
            pltpu.VMEM((CHUNKS_PER_TILE, CHUNK), jnp.int32),
            pltpu.VMEM_SHARED((N_PAD, D_EDGE), jnp.float32),
        ],
        compiler_params=pltpu.CompilerParams(use_tc_tiling_on_sc=False),
    )
    def k(edges_hbm, recv_hbm, zeros_hbm, out_hbm, edges_v, idx_v, acc_sh):
        c = lax.axis_index("c")
        s = lax.axis_index("s")
        wid = s * NC + c

        pltpu.sync_copy(
            zeros_hbm.at[pl.ds(s * ROWS_PER_TILE, ROWS_PER_TILE)],
            acc_sh.at[pl.ds(s * ROWS_PER_TILE, ROWS_PER_TILE)],
        )
        pltpu.sync_copy(
            edges_hbm.at[pl.ds(wid * EDGES_PER_TILE, EDGES_PER_TILE)], edges_v
        )
        pltpu.sync_copy(
            recv_hbm.at[pl.ds(wid * CHUNKS_PER_TILE, CHUNKS_PER_TILE)], idx_v
        )

        plsc.subcore_barrier()

        @pl.loop(0, CHUNKS_PER_TILE)
        def _(j):
            pltpu.sync_copy(
                edges_v.at[pl.ds(j * CHUNK, CHUNK)],
                acc_sh.at[idx_v.at[j]],
                add=True,
            )

        plsc.subcore_barrier()

        pltpu.sync_copy(
            acc_sh.at[pl.ds(s * ROWS_PER_TILE, ROWS_PER_TILE)],
            out_hbm.at[c, pl.ds(s * ROWS_PER_TILE, ROWS_PER_TILE)],
        )

    return k(edges, receivers2d, zeros)


BLK = 1000


def _tc_mlp(p0, p1, nodes, globals_, W_e, W_n, W_g, b2):
    def body(p0_ref, p1_ref, nodes_ref, g_ref, we_ref, wn_ref, wg_ref, b_ref,
             o_ref):
        agg = p0_ref[...] + p1_ref[...]
        acc = jnp.dot(agg, we_ref[...], preferred_element_type=jnp.float32)
        acc += jnp.dot(nodes_ref[...], wn_ref[...],
                       preferred_element_type=jnp.float32)
        acc += jnp.dot(g_ref[...], wg_ref[...],
                       preferred_element_type=jnp.float32)
        o_ref[...] = jnp.maximum(acc + b_ref[...], 0.0)

    return pl.pallas_call(
        body,
        grid=(N_NODES // BLK,),
        in_specs=[
            pl.BlockSpec((BLK, D_EDGE), lambda i: (i, 0)),
            pl.BlockSpec((BLK, D_EDGE), lambda i: (i, 0)),
            pl.BlockSpec((BLK, D_FEAT), lambda i: (i, 0)),
            pl.BlockSpec((1, D_GLOBAL), lambda i: (0, 0)),
            pl.BlockSpec((D_EDGE, D_FEAT), lambda i: (0, 0)),
            pl.BlockSpec((D_FEAT, D_FEAT), lambda i: (0, 0)),
            pl.BlockSpec((D_GLOBAL, D_FEAT), lambda i: (0, 0)),
            pl.BlockSpec((1, D_FEAT), lambda i: (0, 0)),
        ],
        out_specs=pl.BlockSpec((BLK, D_FEAT), lambda i: (i, 0)),
        out_shape=jax.ShapeDtypeStruct((N_NODES, D_FEAT), jnp.float32),
    )(p0, p1, nodes, globals_, W_e, W_n, W_g, b2)


def kernel(nodes, edges, receivers, senders, globals_, W, b):
    del senders
    receivers2d = receivers.reshape(N_EDGES // CHUNK, CHUNK)
    zeros = jnp.zeros((N_PAD, D_EDGE), jnp.float32)
    partials = _sc_segment_sum(edges, receivers2d, zeros)
    partials = partials[:, :N_NODES, :]
    W_e = W[:D_EDGE]
    W_n = W[D_EDGE:D_EDGE + D_FEAT]
    W_g = W[D_EDGE + D_FEAT:]
    return _tc_mlp(partials[0], partials[1], nodes, globals_, W_e, W_n, W_g,
                   b.reshape(1, D_FEAT))


# speedup vs baseline: 4.4569x; 1.0877x over previous
import functools

import jax
import jax.numpy as jnp
from jax import lax
from jax.experimental import pallas as pl
from jax.experimental.pallas import tpu as pltpu
from jax.experimental.pallas import tpu_sc as plsc

N_NODES = 10000
N_EDGES = 160000
D_FEAT = 256
D_EDGE = 16
D_GLOBAL = 128

NC = 2
NS = 16
NW = NC * NS
EDGES_PER_TILE = N_EDGES // NW
CHUNK = 125
CHUNKS_PER_TILE = EDGES_PER_TILE // CHUNK
N_PAD = 10240
ROWS_PER_TILE = N_PAD // NS


def _sc_segment_sum(edges, receivers2d, zeros):
    mesh = plsc.VectorSubcoreMesh(core_axis_name="c", subcore_axis_name="s")

    @functools.partial(
        pl.kernel,
        out_type=jax.ShapeDtypeStruct((NC, N_PAD, D_EDGE), jnp.float32),
        mesh=mesh,
        scratch_types=[
            pltpu.VMEM((EDGES_PER_TILE, D_EDGE), jnp.float32),
            pltpu.VMEM((CHUNKS_PER_TILE, CHUNK), jnp.int32),
            pltpu.VMEM_SHARED((N_PAD, D_EDGE), jnp.float32),
            pltpu.SemaphoreType.DMA,
            pltpu.SemaphoreType.DMA,
        ],
        compiler_params=pltpu.CompilerParams(use_tc_tiling_on_sc=False),
    )
    def k(edges_hbm, recv_hbm, zeros_hbm, out_hbm, edges_v, idx_v, acc_sh,
          stage_sem, scat_sem):
        c = lax.axis_index("c")
        s = lax.axis_index("s")
        wid = s * NC + c

        zero_cp = pltpu.async_copy(
            zeros_hbm.at[pl.ds(s * ROWS_PER_TILE, ROWS_PER_TILE)],
            acc_sh.at[pl.ds(s * ROWS_PER_TILE, ROWS_PER_TILE)],
            stage_sem,
        )
        edges_cp = pltpu.async_copy(
            edges_hbm.at[pl.ds(wid * EDGES_PER_TILE, EDGES_PER_TILE)], edges_v,
            stage_sem,
        )
        idx_cp = pltpu.async_copy(
            recv_hbm.at[pl.ds(wid * CHUNKS_PER_TILE, CHUNKS_PER_TILE)], idx_v,
            stage_sem,
        )
        zero_cp.wait()
        edges_cp.wait()
        idx_cp.wait()

        plsc.subcore_barrier()

        @pl.loop(0, CHUNKS_PER_TILE)
        def _(j):
            pltpu.async_copy(
                edges_v.at[pl.ds(j * CHUNK, CHUNK)],
                acc_sh.at[idx_v.at[j]],
                scat_sem,
                add=True,
            )

        @pl.loop(0, CHUNKS_PER_TILE)
        def _(j):
            pltpu.make_async_copy(
                edges_v.at[pl.ds(j * CHUNK, CHUNK)],
                acc_sh.at[idx_v.at[j]],
                scat_sem,
            ).wait()

        plsc.subcore_barrier()

        pltpu.sync_copy(
            acc_sh.at[pl.ds(s * ROWS_PER_TILE, ROWS_PER_TILE)],
            out_hbm.at[c, pl.ds(s * ROWS_PER_TILE, ROWS_PER_TILE)],
        )

    return k(edges, receivers2d, zeros)


BLK = 1000


def _tc_mlp(partials, nodes, globals_, W_e, W_n, W_g, b2):
    def body(p_ref, nodes_ref, g_ref, we_ref, wn_ref, wg_ref, b_ref, o_ref):
        agg = p_ref[0] + p_ref[1]
        acc = jnp.dot(agg, we_ref[...], preferred_element_type=jnp.float32)
        acc += jnp.dot(nodes_ref[...], wn_ref[...],
                       preferred_element_type=jnp.float32)
        acc += jnp.dot(g_ref[...], wg_ref[...],
                       preferred_element_type=jnp.float32)
        o_ref[...] = jnp.maximum(acc + b_ref[...], 0.0)

    return pl.pallas_call(
        body,
        grid=(N_NODES // BLK,),
        in_specs=[
            pl.BlockSpec((NC, BLK, D_EDGE), lambda i: (0, i, 0)),
            pl.BlockSpec((BLK, D_FEAT), lambda i: (i, 0)),
            pl.BlockSpec((1, D_GLOBAL), lambda i: (0, 0)),
            pl.BlockSpec((D_EDGE, D_FEAT), lambda i: (0, 0)),
            pl.BlockSpec((D_FEAT, D_FEAT), lambda i: (0, 0)),
            pl.BlockSpec((D_GLOBAL, D_FEAT), lambda i: (0, 0)),
            pl.BlockSpec((1, D_FEAT), lambda i: (0, 0)),
        ],
        out_specs=pl.BlockSpec((BLK, D_FEAT), lambda i: (i, 0)),
        out_shape=jax.ShapeDtypeStruct((N_NODES, D_FEAT), jnp.float32),
    )(partials, nodes, globals_, W_e, W_n, W_g, b2)


def kernel(nodes, edges, receivers, senders, globals_, W, b):
    del senders
    receivers2d = receivers.reshape(N_EDGES // CHUNK, CHUNK)
    zeros = jnp.zeros((N_PAD, D_EDGE), jnp.float32)
    partials = _sc_segment_sum(edges, receivers2d, zeros)
    W_e = W[:D_EDGE]
    W_n = W[D_EDGE:D_EDGE + D_FEAT]
    W_g = W[D_EDGE + D_FEAT:]
    return _tc_mlp(partials, nodes, globals_, W_e, W_n, W_g,
                   b.reshape(1, D_FEAT))


# speedup vs baseline: 4.4614x; 1.0010x over previous
import functools

import jax
import jax.numpy as jnp
from jax import lax
from jax.experimental import pallas as pl
from jax.experimental.pallas import tpu as pltpu
from jax.experimental.pallas import tpu_sc as plsc

N_NODES = 10000
N_EDGES = 160000
D_FEAT = 256
D_EDGE = 16
D_GLOBAL = 128

NC = 2
NS = 16
NW = NC * NS
CHUNK = 128
N_CHUNKS = N_EDGES // CHUNK
CHUNKS_FULL = 40
CHUNKS_LAST = N_CHUNKS - (NW - 1) * CHUNKS_FULL
SLAB = CHUNKS_FULL * CHUNK
N_PAD = 10240
ROWS_PER_TILE = N_PAD // NS


def _sc_segment_sum(edges, receivers2d, zeros):
    mesh = plsc.VectorSubcoreMesh(core_axis_name="c", subcore_axis_name="s")

    @functools.partial(
        pl.kernel,
        out_type=jax.ShapeDtypeStruct((NC, N_PAD, D_EDGE), jnp.float32),
        mesh=mesh,
        scratch_types=[
            pltpu.VMEM((SLAB, D_EDGE), jnp.float32),
            pltpu.VMEM((CHUNKS_FULL, CHUNK), jnp.int32),
            pltpu.VMEM_SHARED((N_PAD, D_EDGE), jnp.float32),
            pltpu.SemaphoreType.DMA,
            pltpu.SemaphoreType.DMA,
        ],
        compiler_params=pltpu.CompilerParams(use_tc_tiling_on_sc=False),
    )
    def k(edges_hbm, recv_hbm, zeros_hbm, out_hbm, edges_v, idx_v, acc_sh,
          stage_sem, scat_sem):
        c = lax.axis_index("c")
        s = lax.axis_index("s")
        wid = s * NC + c
        is_last = wid == NW - 1
        n_chunks = jnp.where(is_last, CHUNKS_LAST, CHUNKS_FULL)

        zero_cp = pltpu.async_copy(
            zeros_hbm.at[pl.ds(s * ROWS_PER_TILE, ROWS_PER_TILE)],
            acc_sh.at[pl.ds(s * ROWS_PER_TILE, ROWS_PER_TILE)],
            stage_sem,
        )

        @pl.when(jnp.logical_not(is_last))
        def _():
            e_cp = pltpu.async_copy(
                edges_hbm.at[pl.ds(wid * SLAB, SLAB)], edges_v, stage_sem
            )
            i_cp = pltpu.async_copy(
                recv_hbm.at[pl.ds(wid * CHUNKS_FULL, CHUNKS_FULL)], idx_v,
                stage_sem,
            )
            e_cp.wait()
            i_cp.wait()

        @pl.when(is_last)
        def _():
            e_cp = pltpu.async_copy(
                edges_hbm.at[pl.ds(wid * SLAB, CHUNKS_LAST * CHUNK)],
                edges_v.at[pl.ds(0, CHUNKS_LAST * CHUNK)],
                stage_sem,
            )
            i_cp = pltpu.async_copy(
                recv_hbm.at[pl.ds(wid * CHUNKS_FULL, CHUNKS_LAST)],
                idx_v.at[pl.ds(0, CHUNKS_LAST)],
                stage_sem,
            )
            e_cp.wait()
            i_cp.wait()

        zero_cp.wait()

        plsc.subcore_barrier()

        @pl.loop(0, n_chunks)
        def _(j):
            pltpu.async_copy(
                edges_v.at[pl.ds(j * CHUNK, CHUNK)],
                acc_sh.at[idx_v.at[j]],
                scat_sem,
                add=True,
            )

        @pl.loop(0, n_chunks)
        def _(j):
            pltpu.make_async_copy(
                edges_v.at[pl.ds(j * CHUNK, CHUNK)],
                acc_sh.at[idx_v.at[j]],
                scat_sem,
            ).wait()

        plsc.subcore_barrier()

        pltpu.sync_copy(
            acc_sh.at[pl.ds(s * ROWS_PER_TILE, ROWS_PER_TILE)],
            out_hbm.at[c, pl.ds(s * ROWS_PER_TILE, ROWS_PER_TILE)],
        )

    return k(edges, receivers2d, zeros)


BLK = 1000


def _tc_mlp(partials, nodes, globals_, W_e, W_n, W_g, b2):
    def body(p_ref, nodes_ref, g_ref, we_ref, wn_ref, wg_ref, b_ref, o_ref):
        agg = p_ref[0] + p_ref[1]
        acc = jnp.dot(agg, we_ref[...], preferred_element_type=jnp.float32)
        acc += jnp.dot(nodes_ref[...], wn_ref[...],
                       preferred_element_type=jnp.float32)
        acc += jnp.dot(g_ref[...], wg_ref[...],
                       preferred_element_type=jnp.float32)
        o_ref[...] = jnp.maximum(acc + b_ref[...], 0.0)

    return pl.pallas_call(
        body,
        grid=(N_NODES // BLK,),
        in_specs=[
            pl.BlockSpec((NC, BLK, D_EDGE), lambda i: (0, i, 0)),
            pl.BlockSpec((BLK, D_FEAT), lambda i: (i, 0)),
            pl.BlockSpec((1, D_GLOBAL), lambda i: (0, 0)),
            pl.BlockSpec((D_EDGE, D_FEAT), lambda i: (0, 0)),
            pl.BlockSpec((D_FEAT, D_FEAT), lambda i: (0, 0)),
            pl.BlockSpec((D_GLOBAL, D_FEAT), lambda i: (0, 0)),
            pl.BlockSpec((1, D_FEAT), lambda i: (0, 0)),
        ],
        out_specs=pl.BlockSpec((BLK, D_FEAT), lambda i: (i, 0)),
        out_shape=jax.ShapeDtypeStruct((N_NODES, D_FEAT), jnp.float32),
    )(partials, nodes, globals_, W_e, W_n, W_g, b2)


def kernel(nodes, edges, receivers, senders, globals_, W, b):
    del senders
    receivers2d = receivers.reshape(N_CHUNKS, CHUNK)
    zeros = jnp.zeros((N_PAD, D_EDGE), jnp.float32)
    partials = _sc_segment_sum(edges, receivers2d, zeros)
    W_e = W[:D_EDGE]
    W_n = W[D_EDGE:D_EDGE + D_FEAT]
    W_g = W[D_EDGE + D_FEAT:]
    return _tc_mlp(partials, nodes, globals_, W_e, W_n, W_g,
                   b.reshape(1, D_FEAT))


# speedup vs baseline: 4.4638x; 1.0005x over previous
import functools

import jax
import jax.numpy as jnp
from jax import lax
from jax.experimental import pallas as pl
from jax.experimental.pallas import tpu as pltpu
from jax.experimental.pallas import tpu_sc as plsc

N_NODES = 10000
N_EDGES = 160000
D_FEAT = 256
D_EDGE = 16
D_GLOBAL = 128

NC = 2
NS = 16
NW = NC * NS
CHUNK = 128
N_CHUNKS = N_EDGES // CHUNK
CHUNKS_FULL = 40
CHUNKS_LAST = N_CHUNKS - (NW - 1) * CHUNKS_FULL
SLAB = CHUNKS_FULL * CHUNK
N_PAD = 10240
ROWS_PER_TILE = N_PAD // NS


def _sc_segment_sum(edges, receivers2d, zeros):
    mesh = plsc.VectorSubcoreMesh(core_axis_name="c", subcore_axis_name="s")

    @functools.partial(
        pl.kernel,
        out_type=jax.ShapeDtypeStruct((NC, N_PAD, D_EDGE), jnp.float32),
        mesh=mesh,
        scratch_types=[
            pltpu.VMEM((SLAB, D_EDGE), jnp.float32),
            pltpu.VMEM((SLAB,), jnp.int32),
            pltpu.VMEM_SHARED((N_PAD, D_EDGE), jnp.float32),
            pltpu.SemaphoreType.DMA,
            pltpu.SemaphoreType.DMA,
        ],
        compiler_params=pltpu.CompilerParams(use_tc_tiling_on_sc=False),
    )
    def k(edges_hbm, recv_hbm, zeros_hbm, out_hbm, edges_v, idx_v, acc_sh,
          stage_sem, scat_sem):
        c = lax.axis_index("c")
        s = lax.axis_index("s")
        wid = s * NC + c
        is_last = wid == NW - 1
        n_chunks = jnp.where(is_last, CHUNKS_LAST, CHUNKS_FULL)

        zero_cp = pltpu.async_copy(
            zeros_hbm.at[pl.ds(s * ROWS_PER_TILE, ROWS_PER_TILE)],
            acc_sh.at[pl.ds(s * ROWS_PER_TILE, ROWS_PER_TILE)],
            stage_sem,
        )

        @pl.when(jnp.logical_not(is_last))
        def _():
            e_cp = pltpu.async_copy(
                edges_hbm.at[pl.ds(wid * SLAB, SLAB)], edges_v, stage_sem
            )
            i_cp = pltpu.async_copy(
                recv_hbm.at[pl.ds(wid * SLAB, SLAB)], idx_v, stage_sem
            )
            e_cp.wait()
            i_cp.wait()

        @pl.when(is_last)
        def _():
            e_cp = pltpu.async_copy(
                edges_hbm.at[pl.ds(wid * SLAB, CHUNKS_LAST * CHUNK)],
                edges_v.at[pl.ds(0, CHUNKS_LAST * CHUNK)],
                stage_sem,
            )
            i_cp = pltpu.async_copy(
                recv_hbm.at[pl.ds(wid * SLAB, CHUNKS_LAST * CHUNK)],
                idx_v.at[pl.ds(0, CHUNKS_LAST * CHUNK)],
                stage_sem,
            )
            e_cp.wait()
            i_cp.wait()

        zero_cp.wait()

        plsc.subcore_barrier()

        @pl.loop(0, n_chunks)
        def _(j):
            pltpu.async_copy(
                edges_v.at[pl.ds(j * CHUNK, CHUNK)],
                acc_sh.at[idx_v.at[pl.ds(j * CHUNK, CHUNK)]],
                scat_sem,
                add=True,
            )

        @pl.loop(0, n_chunks)
        def _(j):
            pltpu.make_async_copy(
                edges_v.at[pl.ds(j * CHUNK, CHUNK)],
                acc_sh.at[idx_v.at[pl.ds(j * CHUNK, CHUNK)]],
                scat_sem,
            ).wait()

        plsc.subcore_barrier()

        pltpu.sync_copy(
            acc_sh.at[pl.ds(s * ROWS_PER_TILE, ROWS_PER_TILE)],
            out_hbm.at[c, pl.ds(s * ROWS_PER_TILE, ROWS_PER_TILE)],
        )

    return k(edges, receivers2d, zeros)


BLK = 1000


def _tc_mlp(partials, nodes, globals_, W_e, W_n, W_g, b2):
    def body(p_ref, nodes_ref, g_ref, we_ref, wn_ref, wg_ref, b_ref, o_ref):
        agg = p_ref[0] + p_ref[1]
        acc = jnp.dot(agg, we_ref[...], preferred_element_type=jnp.float32)
        acc += jnp.dot(nodes_ref[...], wn_ref[...],
                       preferred_element_type=jnp.float32)
        acc += jnp.dot(g_ref[...], wg_ref[...],
                       preferred_element_type=jnp.float32)
        o_ref[...] = jnp.maximum(acc + b_ref[...], 0.0)

    return pl.pallas_call(
        body,
        grid=(N_NODES // BLK,),
        in_specs=[
            pl.BlockSpec((NC, BLK, D_EDGE), lambda i: (0, i, 0)),
            pl.BlockSpec((BLK, D_FEAT), lambda i: (i, 0)),
            pl.BlockSpec((1, D_GLOBAL), lambda i: (0, 0)),
            pl.BlockSpec((D_EDGE, D_FEAT), lambda i: (0, 0)),
            pl.BlockSpec((D_FEAT, D_FEAT), lambda i: (0, 0)),
            pl.BlockSpec((D_GLOBAL, D_FEAT), lambda i: (0, 0)),
            pl.BlockSpec((1, D_FEAT), lambda i: (0, 0)),
        ],
        out_specs=pl.BlockSpec((BLK, D_FEAT), lambda i: (i, 0)),
        out_shape=jax.ShapeDtypeStruct((N_NODES, D_FEAT), jnp.float32),
    )(partials, nodes, globals_, W_e, W_n, W_g, b2)


def kernel(nodes, edges, receivers, senders, globals_, W, b):
    del senders
    zeros = jnp.zeros((N_PAD, D_EDGE), jnp.float32)
    partials = _sc_segment_sum(edges, receivers, zeros)
    W_e = W[:D_EDGE]
    W_n = W[D_EDGE:D_EDGE + D_FEAT]
    W_g = W[D_EDGE + D_FEAT:]
    return _tc_mlp(partials, nodes, globals_, W_e, W_n, W_g,
                   b.reshape(1, D_FEAT))
